# baseline (speedup 1.0000x reference)
import jax
import jax.numpy as jnp
from jax.experimental import pallas as pl
from jax.experimental.pallas import tpu as pltpu

_SLOTS = 4
_CHUNK = 128


def _dot(a, b, dims):
    return jax.lax.dot_general(a, b, (dims, ((), ())),
                               preferred_element_type=jnp.float32)


def _gnn_kernel(x_hbm, mask_ref, w1_ref, b1_ref, wl1_ref, bl1_ref, wr1_ref,
                wl2_ref, bl2_ref, wr2_ref, out_ref, xbuf, sem):
    bsz, n, hdim = x_hbm.shape
    per_b = n // _CHUNK
    nchunks = bsz * per_b
    copies = [
        pltpu.make_async_copy(
            x_hbm.at[k // per_b,
                     pl.ds((k % per_b) * _CHUNK, _CHUNK), :],
            xbuf.at[k % _SLOTS], sem.at[k % _SLOTS])
        for k in range(nchunks)
    ]
    for k in range(min(_SLOTS, nchunks)):
        copies[k].start()

    b1 = b1_ref[...].reshape(1, b1_ref.shape[0])
    h_parts = []
    for k in range(nchunks):
        copies[k].wait()
        h_parts.append(_dot(xbuf[k % _SLOTS], w1_ref[...], (((1,), (1,)))))
        if k + _SLOTS < nchunks:
            copies[k + _SLOTS].start()
    h_all = jnp.concatenate(h_parts, axis=0) + b1

    bl1 = bl1_ref[...].reshape(1, bl1_ref.shape[0])
    bl2 = bl2_ref[...].reshape(1, 1)
    inv_cnt = 1.0 / (n - 1)

    nrm2 = jnp.sum(h_all * h_all, axis=1, keepdims=True)
    rn = 1.0 / jnp.maximum(jnp.sqrt(nrm2), 1e-8)
    c = nrm2 * (rn * rn)

    aa = []
    agg1_parts = []
    for i in range(bsz):
        h = h_all[i * n:(i + 1) * n]
        rni = rn[i * n:(i + 1) * n]
        g = _dot(h, h, (((1,), (1,))))
        a = (g * rni) * rni.reshape(1, n)
        aa.append(a)
        agg1_parts.append(_dot(a, h, (((1,), (0,)))))
    agg1 = (jnp.concatenate(agg1_parts, axis=0) - c * h_all) * inv_cnt

    o1_all = jnp.maximum(
        _dot(agg1, wl1_ref[...], (((1,), (1,))))
        + _dot(h_all, wr1_ref[...], (((1,), (1,))))
        + bl1, 0.0)

    agg2_parts = [
        _dot(aa[i], o1_all[i * n:(i + 1) * n], (((1,), (0,))))
        for i in range(bsz)
    ]
    agg2 = (jnp.concatenate(agg2_parts, axis=0) - c * o1_all) * inv_cnt
    z = (_dot(wl2_ref[...], agg2, (((1,), (1,))))
         + _dot(wr2_ref[...], o1_all, (((1,), (1,))))
         + bl2)
    out_ref[...] = jax.nn.sigmoid(z.reshape(bsz, n)) * mask_ref[...]


@jax.jit
def kernel(x, mask_cls, W1, b1, Wl1, bl1, Wr1, Wl2, bl2, Wr2):
    B, N, H = x.shape
    vmem = pl.BlockSpec(memory_space=pltpu.MemorySpace.VMEM)
    return pl.pallas_call(
        _gnn_kernel,
        in_specs=[pl.BlockSpec(memory_space=pltpu.MemorySpace.HBM),
                  vmem, vmem, vmem, vmem, vmem, vmem, vmem, vmem, vmem],
        out_specs=vmem,
        out_shape=jax.ShapeDtypeStruct((B, N), jnp.float32),
        scratch_shapes=[
            pltpu.VMEM((_SLOTS, _CHUNK, H), jnp.float32),
            pltpu.SemaphoreType.DMA((_SLOTS,)),
        ],
    )(x, mask_cls, W1, b1, Wl1, bl1, Wr1, Wl2, bl2, Wr2)
